# baseline (speedup 1.0000x reference)
import functools

import numpy as np
import jax
import jax.numpy as jnp
from jax import lax
from jax.experimental import pallas as pl
from jax.experimental.pallas import tpu as pltpu
from jax.experimental.pallas import tpu_sc as plsc

_NC = 2
_NS = 16
_NW = _NC * _NS
_LANES = 16

_CHUNK = 16


@functools.lru_cache(maxsize=None)
def _pe_table(seq_len: int, d_model: int):
    pos = np.arange(seq_len, dtype=np.float32)[:, None]
    i = np.arange(0, d_model, 2, dtype=np.float32)[None, :]
    angle = pos / np.power(10000.0, i / d_model)
    pe = np.zeros((seq_len, d_model), dtype=np.float32)
    pe[:, 0::2] = np.sin(angle)
    pe[:, 1::2] = np.cos(angle)
    return jnp.asarray(pe)


def _emb_body(nchunk, chunk, pos_per_w, d_model, seq_len,
              tab_hbm, idx_hbm, pe_hbm, out_hbm,
              idx_v, pe_res, rows0, rows1,
              gsem0, gsem1, osem0, osem1):
    wid = lax.axis_index("s") * _NC + lax.axis_index("c")
    vecs_per_row = d_model // _LANES
    rows_per_batch = pos_per_w // chunk

    pltpu.sync_copy(idx_hbm.at[wid], idx_v)
    pltpu.sync_copy(pe_hbm.at[pl.ds(wid * pos_per_w, pos_per_w)], pe_res)

    rbufs = (rows0, rows1)
    gsems = (gsem0, gsem1)
    osems = (osem0, osem1)

    hg = [None] * nchunk
    hout = [None] * nchunk

    def issue_gather(c):
        hg[c] = pltpu.async_copy(tab_hbm.at[idx_v.at[c]], rbufs[c & 1],
                                 gsems[c & 1])

    issue_gather(0)
    for c in range(nchunk):
        buf = c & 1
        rows = rbufs[buf]
        hg[c].wait()
        if c + 1 < nchunk:
            if c - 1 >= 0:
                hout[c - 1].wait()
            issue_gather(c + 1)

        b, q = divmod(c, rows_per_batch)
        q *= chunk

        def add_pe(r, _, rows=rows, q=q):
            for j in range(vecs_per_row):
                sl = pl.ds(j * _LANES, _LANES)
                plsc.addupdate(rows.at[r, sl], pe_res[q + r, sl])
            return _

        lax.fori_loop(0, chunk, add_pe, 0)

        out_base = b * seq_len + wid * pos_per_w + q
        hout[c] = pltpu.async_copy(rows, out_hbm.at[pl.ds(out_base, chunk)],
                                   osems[buf])
    hout[nchunk - 2].wait()
    hout[nchunk - 1].wait()


def kernel(x, tok_table):
    batch, seq_len = x.shape
    vocab, d_model = tok_table.shape
    assert seq_len % _NW == 0 and d_model % _LANES == 0
    pos_per_w = seq_len // _NW
    assert pos_per_w % _CHUNK == 0
    nchunk = batch * (pos_per_w // _CHUNK)

    xi = x.astype(jnp.int32)
    xr = jnp.transpose(xi.reshape(batch, _NW, pos_per_w), (1, 0, 2))
    xr = xr.reshape(_NW, nchunk, _CHUNK)

    pe = _pe_table(seq_len, d_model)

    mesh = plsc.VectorSubcoreMesh(core_axis_name="c", subcore_axis_name="s")
    body = functools.partial(_emb_body, nchunk, _CHUNK, pos_per_w, d_model,
                             seq_len)
    emb = pl.kernel(
        body,
        mesh=mesh,
        out_type=jax.ShapeDtypeStruct((batch * seq_len, d_model), jnp.float32),
        scratch_types=[
            pltpu.VMEM((nchunk, _CHUNK), jnp.int32),
            pltpu.VMEM((pos_per_w, d_model), jnp.float32),
            pltpu.VMEM((_CHUNK, d_model), jnp.float32),
            pltpu.VMEM((_CHUNK, d_model), jnp.float32),
            pltpu.SemaphoreType.DMA,
            pltpu.SemaphoreType.DMA,
            pltpu.SemaphoreType.DMA,
            pltpu.SemaphoreType.DMA,
        ],
    )
    out = emb(tok_table, xr, pe)
    return out.reshape(batch, seq_len, d_model)


# speedup vs baseline: 1.4095x; 1.4095x over previous
import functools

import numpy as np
import jax
import jax.numpy as jnp
from jax import lax
from jax.experimental import pallas as pl
from jax.experimental.pallas import tpu as pltpu
from jax.experimental.pallas import tpu_sc as plsc

_NC = 2
_NS = 16
_NW = _NC * _NS
_LANES = 16

_CHUNK = 16
_NBUF = 3


@functools.lru_cache(maxsize=None)
def _pe_table(seq_len: int, d_model: int):
    pos = np.arange(seq_len, dtype=np.float32)[:, None]
    i = np.arange(0, d_model, 2, dtype=np.float32)[None, :]
    angle = pos / np.power(10000.0, i / d_model)
    pe = np.zeros((seq_len, d_model), dtype=np.float32)
    pe[:, 0::2] = np.sin(angle)
    pe[:, 1::2] = np.cos(angle)
    return jnp.asarray(pe)


def _emb_body(batch, pos_per_w, d_model, seq_len,
              tab_hbm, idx_hbm, pe_hbm, out_hbm,
              idx_v, pe_res, *bufs_and_sems):
    cpb = pos_per_w // _CHUNK
    rbufs = bufs_and_sems[:_NBUF]
    gsems = bufs_and_sems[_NBUF:2 * _NBUF]
    osems = bufs_and_sems[2 * _NBUF:3 * _NBUF]
    isems = bufs_and_sems[3 * _NBUF:3 * _NBUF + batch]
    psems = bufs_and_sems[3 * _NBUF + batch:]

    wid = lax.axis_index("s") * _NC + lax.axis_index("c")
    vecs_per_row = d_model // _LANES
    nchunk = batch * cpb
    pos0 = wid * pos_per_w

    hidx = [pltpu.async_copy(idx_hbm.at[b, pl.ds(pos0, pos_per_w)],
                             idx_v.at[b], isems[b]) for b in range(batch)]
    hpe = [pltpu.async_copy(pe_hbm.at[pl.ds(pos0 + p * _CHUNK, _CHUNK)],
                            pe_res.at[pl.ds(p * _CHUNK, _CHUNK)], psems[p])
           for p in range(cpb)]
    pe_ready = [False] * cpb
    idx_ready = [False] * batch

    hg = [None] * nchunk
    hout = [None] * nchunk

    def issue_gather(c):
        b, q = divmod(c, cpb)
        if not idx_ready[b]:
            hidx[b].wait()
            idx_ready[b] = True
        hg[c] = pltpu.async_copy(
            tab_hbm.at[idx_v.at[b, pl.ds(q * _CHUNK, _CHUNK)]],
            rbufs[c % _NBUF], gsems[c % _NBUF])

    for c in range(min(2, nchunk)):
        issue_gather(c)

    for c in range(nchunk):
        buf = c % _NBUF
        rows = rbufs[buf]
        hg[c].wait()
        if c + 2 < nchunk:
            if c - 1 >= 0:
                hout[c - 1].wait()
            issue_gather(c + 2)

        b, p = divmod(c, cpb)
        if not pe_ready[p]:
            hpe[p].wait()
            pe_ready[p] = True
        q = p * _CHUNK

        def add_pe(r, carry, rows=rows, q=q):
            for j in range(vecs_per_row):
                sl = pl.ds(j * _LANES, _LANES)
                plsc.addupdate(rows.at[r, sl], pe_res[q + r, sl])
            return carry

        lax.fori_loop(0, _CHUNK, add_pe, 0)

        out_base = b * seq_len + pos0 + q
        hout[c] = pltpu.async_copy(rows, out_hbm.at[pl.ds(out_base, _CHUNK)],
                                   osems[buf])
    for c in range(max(0, nchunk - 2), nchunk):
        hout[c].wait()


def kernel(x, tok_table):
    batch, seq_len = x.shape
    vocab, d_model = tok_table.shape
    assert seq_len % _NW == 0 and d_model % _LANES == 0
    pos_per_w = seq_len // _NW
    assert pos_per_w % _CHUNK == 0

    xi = x.astype(jnp.int32)
    pe = _pe_table(seq_len, d_model)

    mesh = plsc.VectorSubcoreMesh(core_axis_name="c", subcore_axis_name="s")
    body = functools.partial(_emb_body, batch, pos_per_w, d_model, seq_len)
    emb = pl.kernel(
        body,
        mesh=mesh,
        out_type=jax.ShapeDtypeStruct((batch * seq_len, d_model), jnp.float32),
        scratch_types=[
            pltpu.VMEM((batch, pos_per_w), jnp.int32),
            pltpu.VMEM((pos_per_w, d_model), jnp.float32),
        ] + [pltpu.VMEM((_CHUNK, d_model), jnp.float32)] * _NBUF
          + [pltpu.SemaphoreType.DMA] * (2 * _NBUF)
          + [pltpu.SemaphoreType.DMA] * (batch + pos_per_w // _CHUNK),
    )
    out = emb(tok_table, xi, pe)
    return out.reshape(batch, seq_len, d_model)
